# baseline (speedup 1.0000x reference)
import functools

import jax
import jax.numpy as jnp
from jax import lax
from jax.experimental import pallas as pl
from jax.experimental.pallas import tpu as pltpu
from jax.experimental.pallas import tpu_sc as plsc

B = 16384
NC = 2
NS = 16
NW = NC * NS
BPW = B // NW
CHUNK = 128
NCHUNK = BPW // CHUNK

D_MLP = 32
D_MF = 16
BLK = 2048


def _sc_gather(user_idx, item_idx, U_mlp, I_mlp, U_mf, I_mf):
    uidx = user_idx.reshape(NW, NCHUNK, CHUNK)
    iidx = item_idx.reshape(NW, NCHUNK, CHUNK)
    mesh = plsc.VectorSubcoreMesh(core_axis_name="c", subcore_axis_name="s")

    @functools.partial(
        pl.kernel,
        mesh=mesh,
        compiler_params=pltpu.CompilerParams(use_tc_tiling_on_sc=False),
        out_type=[
            jax.ShapeDtypeStruct((B, D_MLP), jnp.float32),
            jax.ShapeDtypeStruct((B, D_MLP), jnp.float32),
            jax.ShapeDtypeStruct((B, D_MF), jnp.float32),
            jax.ShapeDtypeStruct((B, D_MF), jnp.float32),
        ],
        scratch_types=[
            pltpu.VMEM((NCHUNK, CHUNK), jnp.int32),
            pltpu.VMEM((NCHUNK, CHUNK), jnp.int32),
            pltpu.VMEM((BPW, D_MLP), jnp.float32),
            pltpu.VMEM((BPW, D_MLP), jnp.float32),
            pltpu.VMEM((BPW, D_MF), jnp.float32),
            pltpu.VMEM((BPW, D_MF), jnp.float32),
            pltpu.SemaphoreType.DMA,
        ],
    )
    def k(uidx_h, iidx_h, umlp_h, imlp_h, umf_h, imf_h,
          ou_mlp, oi_mlp, ou_mf, oi_mf,
          uv, iv, bu_mlp, bi_mlp, bu_mf, bi_mf, sem):
        wid = lax.axis_index("s") * NC + lax.axis_index("c")
        base = wid * BPW
        pltpu.sync_copy(uidx_h.at[wid], uv)
        pltpu.sync_copy(iidx_h.at[wid], iv)
        copies = []
        for j in range(NCHUNK):
            sl = pl.ds(j * CHUNK, CHUNK)
            copies.append(pltpu.async_copy(umlp_h.at[uv.at[j]], bu_mlp.at[sl], sem))
            copies.append(pltpu.async_copy(imlp_h.at[iv.at[j]], bi_mlp.at[sl], sem))
            copies.append(pltpu.async_copy(umf_h.at[uv.at[j]], bu_mf.at[sl], sem))
            copies.append(pltpu.async_copy(imf_h.at[iv.at[j]], bi_mf.at[sl], sem))
        for c in copies:
            c.wait()
        pltpu.sync_copy(bu_mlp, ou_mlp.at[pl.ds(base, BPW)])
        pltpu.sync_copy(bi_mlp, oi_mlp.at[pl.ds(base, BPW)])
        pltpu.sync_copy(bu_mf, ou_mf.at[pl.ds(base, BPW)])
        pltpu.sync_copy(bi_mf, oi_mf.at[pl.ds(base, BPW)])

    return k(uidx, iidx, U_mlp, I_mlp, U_mf, I_mf)


def _mlp_body(u_mlp, i_mlp, u_mf, i_mf, w1u, w1i, b1, w2, b2, w3, b3,
              wo3, womf, bo, out):
    h = jnp.dot(u_mlp[...], w1u[...], preferred_element_type=jnp.float32)
    h = h + jnp.dot(i_mlp[...], w1i[...], preferred_element_type=jnp.float32)
    h = jnp.maximum(h + b1[...], 0.0)
    h = jnp.maximum(jnp.dot(h, w2[...], preferred_element_type=jnp.float32) + b2[...], 0.0)
    h = jnp.maximum(jnp.dot(h, w3[...], preferred_element_type=jnp.float32) + b3[...], 0.0)
    z = jnp.dot(h, wo3[...], preferred_element_type=jnp.float32)
    z = z + jnp.dot(u_mf[...] * i_mf[...], womf[...], preferred_element_type=jnp.float32)
    out[...] = jax.nn.sigmoid(z + bo[...])


def _tc_mlp(gu_mlp, gi_mlp, gu_mf, gi_mf, W1, b1, W2, b2, W3, b3, Wo, bo):
    w1u = W1[:D_MLP]
    w1i = W1[D_MLP:]
    wo3 = Wo[:8]
    womf = Wo[8:]
    b1r = b1.reshape(1, -1)
    b2r = b2.reshape(1, -1)
    b3r = b3.reshape(1, -1)
    bor = bo.reshape(1, -1)

    def full(a):
        return pl.BlockSpec(a.shape, lambda i: (0, 0))

    batch = lambda d: pl.BlockSpec((BLK, d), lambda i: (i, 0))
    return pl.pallas_call(
        _mlp_body,
        grid=(B // BLK,),
        in_specs=[
            batch(D_MLP), batch(D_MLP), batch(D_MF), batch(D_MF),
            full(w1u), full(w1i), full(b1r), full(W2), full(b2r),
            full(W3), full(b3r), full(wo3), full(womf), full(bor),
        ],
        out_specs=pl.BlockSpec((BLK, 1), lambda i: (i, 0)),
        out_shape=jax.ShapeDtypeStruct((B, 1), jnp.float32),
    )(gu_mlp, gi_mlp, gu_mf, gi_mf, w1u, w1i, b1r, W2, b2r, W3, b3r,
      wo3, womf, bor)


def kernel(user_indices, item_indices, U_mlp, I_mlp, U_mf, I_mf,
           W1, b1, W2, b2, W3, b3, Wo, bo):
    gu_mlp, gi_mlp, gu_mf, gi_mf = _sc_gather(
        user_indices, item_indices, U_mlp, I_mlp, U_mf, I_mf)
    return _tc_mlp(gu_mlp, gi_mlp, gu_mf, gi_mf,
                   W1, b1, W2, b2, W3, b3, Wo, bo)


# speedup vs baseline: 1.9423x; 1.9423x over previous
import functools

import jax
import jax.numpy as jnp
from jax import lax
from jax.experimental import pallas as pl
from jax.experimental.pallas import tpu as pltpu
from jax.experimental.pallas import tpu_sc as plsc

B = 16384
NC = 2
NS = 16
NW = NC * NS
BPW = B // NW
CH = 16
NCH = BPW // CH

D_MLP = 32
D_MF = 16
BLK = 2048


def _sc_gather(user_idx, item_idx, U_mlp, I_mlp, U_mf, I_mf):
    uidx = user_idx.reshape(NW, BPW // 128, 128)
    iidx = item_idx.reshape(NW, BPW // 128, 128)
    umlp3 = U_mlp.reshape(-1, 8, D_MLP)
    imlp3 = I_mlp.reshape(-1, 8, D_MLP)
    umf3 = U_mf.reshape(-1, 8, D_MF)
    imf3 = I_mf.reshape(-1, 8, D_MF)
    mesh = plsc.VectorSubcoreMesh(core_axis_name="c", subcore_axis_name="s")

    @functools.partial(
        pl.kernel,
        mesh=mesh,
        compiler_params=pltpu.CompilerParams(use_tc_tiling_on_sc=True),
        out_type=jax.ShapeDtypeStruct((B, 128), jnp.float32),
        scratch_types=[
            pltpu.VMEM((BPW // 128, 128), jnp.int32),
            pltpu.VMEM((BPW // 128, 128), jnp.int32),
            pltpu.VMEM((CH, 8, D_MLP), jnp.float32),
            pltpu.VMEM((CH, 8, D_MLP), jnp.float32),
            pltpu.VMEM((CH, 8, D_MF), jnp.float32),
            pltpu.VMEM((CH, 8, D_MF), jnp.float32),
            pltpu.VMEM((CH, 128), jnp.float32),
            pltpu.SemaphoreType.DMA,
        ],
    )
    def k(uidx_h, iidx_h, umlp_h, imlp_h, umf_h, imf_h, out_h,
          uv, iv, gu_mlp, gi_mlp, gu_mf, gi_mf, slab, sem):
        wid = lax.axis_index("s") * NC + lax.axis_index("c")
        base = wid * BPW
        pltpu.sync_copy(uidx_h.at[wid], uv)
        pltpu.sync_copy(iidx_h.at[wid], iv)

        def chunk(c, _):
            j = c // 8
            lb = (c % 8) * CH
            uvec = uv[j, pl.ds(lb, CH)]
            ivec = iv[j, pl.ds(lb, CH)]
            ug = lax.shift_right_logical(uvec, 3)
            ig = lax.shift_right_logical(ivec, 3)
            for l in range(CH):
                ugl = ug[l]
                igl = ig[l]
                pltpu.async_copy(umlp_h.at[pl.ds(ugl, 1)],
                                 gu_mlp.at[pl.ds(l, 1)], sem)
                pltpu.async_copy(imlp_h.at[pl.ds(igl, 1)],
                                 gi_mlp.at[pl.ds(l, 1)], sem)
                pltpu.async_copy(umf_h.at[pl.ds(ugl, 1)],
                                 gu_mf.at[pl.ds(l, 1)], sem)
                pltpu.async_copy(imf_h.at[pl.ds(igl, 1)],
                                 gi_mf.at[pl.ds(l, 1)], sem)
            pltpu.make_async_copy(
                umlp_h.at[pl.ds(0, CH)], gu_mlp, sem).wait()
            pltpu.make_async_copy(
                imlp_h.at[pl.ds(0, CH)], gi_mlp, sem).wait()
            pltpu.make_async_copy(
                umf_h.at[pl.ds(0, CH)], gu_mf, sem).wait()
            pltpu.make_async_copy(
                imf_h.at[pl.ds(0, CH)], gi_mf, sem).wait()
            usub = lax.bitwise_and(uvec, 7)
            isub = lax.bitwise_and(ivec, 7)
            for l in range(CH):
                us = usub[l]
                isv = isub[l]
                slab[l, pl.ds(0, 16)] = gu_mlp[l, us, pl.ds(0, 16)]
                slab[l, pl.ds(16, 16)] = gu_mlp[l, us, pl.ds(16, 16)]
                slab[l, pl.ds(32, 16)] = gi_mlp[l, isv, pl.ds(0, 16)]
                slab[l, pl.ds(48, 16)] = gi_mlp[l, isv, pl.ds(16, 16)]
                slab[l, pl.ds(64, 16)] = gu_mf[l, us, pl.ds(0, 16)]
                slab[l, pl.ds(80, 16)] = gi_mf[l, isv, pl.ds(0, 16)]
            row0 = pl.multiple_of(base + c * CH, 8)
            pltpu.sync_copy(slab, out_h.at[pl.ds(row0, CH)])
            return 0

        lax.fori_loop(0, NCH, chunk, 0)

    return k(uidx, iidx, umlp3, imlp3, umf3, imf3)


def _mlp_body(x, w1, b1, w2, b2, w3, b3, wo3, womf, bo, out):
    xb = x[...]
    h = jnp.dot(xb[:, 0:64], w1[...], preferred_element_type=jnp.float32)
    h = jnp.maximum(h + b1[...], 0.0)
    h = jnp.maximum(jnp.dot(h, w2[...], preferred_element_type=jnp.float32) + b2[...], 0.0)
    h = jnp.maximum(jnp.dot(h, w3[...], preferred_element_type=jnp.float32) + b3[...], 0.0)
    z = jnp.dot(h, wo3[...], preferred_element_type=jnp.float32)
    mf = xb[:, 64:80] * xb[:, 80:96]
    z = z + jnp.dot(mf, womf[...], preferred_element_type=jnp.float32)
    out[...] = jax.nn.sigmoid(z + bo[...])


def _tc_mlp(x, W1, b1, W2, b2, W3, b3, Wo, bo):
    wo3 = Wo[:8]
    womf = Wo[8:]
    b1r = b1.reshape(1, -1)
    b2r = b2.reshape(1, -1)
    b3r = b3.reshape(1, -1)
    bor = bo.reshape(1, -1)

    def full(a):
        return pl.BlockSpec(a.shape, lambda i: (0, 0))

    return pl.pallas_call(
        _mlp_body,
        grid=(B // BLK,),
        in_specs=[
            pl.BlockSpec((BLK, 128), lambda i: (i, 0)),
            full(W1), full(b1r), full(W2), full(b2r),
            full(W3), full(b3r), full(wo3), full(womf), full(bor),
        ],
        out_specs=pl.BlockSpec((BLK, 1), lambda i: (i, 0)),
        out_shape=jax.ShapeDtypeStruct((B, 1), jnp.float32),
    )(x, W1, b1r, W2, b2r, W3, b3r, wo3, womf, bor)


def kernel(user_indices, item_indices, U_mlp, I_mlp, U_mf, I_mf,
           W1, b1, W2, b2, W3, b3, Wo, bo):
    x = _sc_gather(user_indices, item_indices, U_mlp, I_mlp, U_mf, I_mf)
    return _tc_mlp(x, W1, b1, W2, b2, W3, b3, Wo, bo)
